# baseline (device time: 20653 ns/iter reference)
import jax
import jax.numpy as jnp
from jax import lax
from jax.experimental import pallas as pl
from jax.experimental.pallas import tpu as pltpu

BM = 256


def kernel(x, dy, gamma):
    del gamma
    m, d = x.shape
    half_steps = (m // 2) // BM

    def body(off_ref, x_ref, dy_ref, out_ref, acc_ref, recv_ref,
             send_sems, recv_sems):
        step = pl.program_id(0)
        my_x = lax.axis_index("x")
        my_y = lax.axis_index("y")
        peers = [
            (my_x, 1 - my_y),
            (1 - my_x, my_y),
            (1 - my_x, 1 - my_y),
        ]

        @pl.when(step == 0)
        def _init():
            acc_ref[...] = jnp.zeros_like(acc_ref)
            barrier = pltpu.get_barrier_semaphore()
            for nbr in peers:
                pl.semaphore_signal(
                    barrier,
                    inc=1,
                    device_id=nbr,
                    device_id_type=pl.DeviceIdType.MESH,
                )
            pl.semaphore_wait(barrier, 3)

        xb = x_ref[...]
        dyb = dy_ref[...]
        mu = jnp.mean(xb, axis=1, keepdims=True)
        xc = xb - mu
        var = jnp.mean(xc * xc, axis=1, keepdims=True)
        rstd = lax.rsqrt(var + 1e-5)
        xhat = xc * rstd
        acc_ref[0, :] = acc_ref[0, :] + jnp.sum(dyb * xhat, axis=0)
        acc_ref[1, :] = acc_ref[1, :] + jnp.sum(dyb, axis=0)

        @pl.when(step == half_steps - 1)
        def _allreduce():
            rdmas = []
            for k, nbr in enumerate(peers):
                rdma = pltpu.make_async_remote_copy(
                    src_ref=acc_ref,
                    dst_ref=recv_ref.at[k],
                    send_sem=send_sems.at[k],
                    recv_sem=recv_sems.at[k],
                    device_id=nbr,
                    device_id_type=pl.DeviceIdType.MESH,
                )
                rdma.start()
                rdmas.append(rdma)
            for rdma in rdmas:
                rdma.wait()
            out_ref[...] = (
                acc_ref[...] + recv_ref[0] + recv_ref[1] + recv_ref[2]
            )

    grid_spec = pltpu.PrefetchScalarGridSpec(
        num_scalar_prefetch=1,
        grid=(half_steps,),
        in_specs=[
            pl.BlockSpec((BM, d), lambda i, off: (off[0] + i, 0)),
            pl.BlockSpec((BM, d), lambda i, off: (off[0] + i, 0)),
        ],
        out_specs=pl.BlockSpec((2, d), lambda i, off: (0, 0)),
        scratch_shapes=[
            pltpu.VMEM((2, d), jnp.float32),
            pltpu.VMEM((3, 2, d), jnp.float32),
            pltpu.SemaphoreType.DMA((3,)),
            pltpu.SemaphoreType.DMA((3,)),
        ],
    )

    offset = (lax.axis_index("y") * half_steps).astype(jnp.int32).reshape(1)

    return pl.pallas_call(
        body,
        grid_spec=grid_spec,
        out_shape=jax.ShapeDtypeStruct((2, d), jnp.float32),
        compiler_params=pltpu.CompilerParams(
            collective_id=0, vmem_limit_bytes=96 * 1024 * 1024
        ),
    )(offset, x, dy)


# device time: 19861 ns/iter; 1.0399x vs baseline; 1.0399x over previous
import jax
import jax.numpy as jnp
from jax import lax
from jax.experimental import pallas as pl
from jax.experimental.pallas import tpu as pltpu

BM = 512


def kernel(x, dy, gamma):
    del gamma
    m, d = x.shape
    half_steps = (m // 2) // BM

    def body(off_ref, x_ref, dy_ref, out_ref, acc_ref, recv_ref,
             send_sems, recv_sems):
        step = pl.program_id(0)
        my_x = lax.axis_index("x")
        my_y = lax.axis_index("y")
        peers = [
            (my_x, 1 - my_y),
            (1 - my_x, my_y),
            (1 - my_x, 1 - my_y),
        ]

        @pl.when(step == 0)
        def _init():
            acc_ref[...] = jnp.zeros_like(acc_ref)
            barrier = pltpu.get_barrier_semaphore()
            for nbr in peers:
                pl.semaphore_signal(
                    barrier,
                    inc=1,
                    device_id=nbr,
                    device_id_type=pl.DeviceIdType.MESH,
                )
            pl.semaphore_wait(barrier, 3)

        xb = x_ref[...]
        dyb = dy_ref[...]
        mu = jnp.mean(xb, axis=1, keepdims=True)
        xc = xb - mu
        var = jnp.mean(xc * xc, axis=1, keepdims=True)
        rstd = lax.rsqrt(var + 1e-5)
        xhat = xc * rstd
        acc_ref[0, :] = acc_ref[0, :] + jnp.sum(dyb * xhat, axis=0)
        acc_ref[1, :] = acc_ref[1, :] + jnp.sum(dyb, axis=0)

        @pl.when(step == half_steps - 1)
        def _allreduce():
            rdmas = []
            for k, nbr in enumerate(peers):
                rdma = pltpu.make_async_remote_copy(
                    src_ref=acc_ref,
                    dst_ref=recv_ref.at[k],
                    send_sem=send_sems.at[k],
                    recv_sem=recv_sems.at[k],
                    device_id=nbr,
                    device_id_type=pl.DeviceIdType.MESH,
                )
                rdma.start()
                rdmas.append(rdma)
            for rdma in rdmas:
                rdma.wait()
            out_ref[...] = (
                acc_ref[...] + recv_ref[0] + recv_ref[1] + recv_ref[2]
            )

    grid_spec = pltpu.PrefetchScalarGridSpec(
        num_scalar_prefetch=1,
        grid=(half_steps,),
        in_specs=[
            pl.BlockSpec((BM, d), lambda i, off: (off[0] + i, 0)),
            pl.BlockSpec((BM, d), lambda i, off: (off[0] + i, 0)),
        ],
        out_specs=pl.BlockSpec((2, d), lambda i, off: (0, 0)),
        scratch_shapes=[
            pltpu.VMEM((2, d), jnp.float32),
            pltpu.VMEM((3, 2, d), jnp.float32),
            pltpu.SemaphoreType.DMA((3,)),
            pltpu.SemaphoreType.DMA((3,)),
        ],
    )

    offset = (lax.axis_index("y") * half_steps).astype(jnp.int32).reshape(1)

    return pl.pallas_call(
        body,
        grid_spec=grid_spec,
        out_shape=jax.ShapeDtypeStruct((2, d), jnp.float32),
        compiler_params=pltpu.CompilerParams(
            collective_id=0, vmem_limit_bytes=96 * 1024 * 1024
        ),
    )(offset, x, dy)
